# baseline (speedup 1.0000x reference)
import functools

import jax
import jax.numpy as jnp
from jax import lax
from jax.experimental import pallas as pl
from jax.experimental.pallas import tpu as pltpu
from jax.experimental.pallas import tpu_sc as plsc

_NUM_CORES = 2
_NUM_SUBCORES = 16
_NW = _NUM_CORES * _NUM_SUBCORES


_HB = 8


def _transpose_body(x_ref, eye_ref, o_ref):
    W = x_ref.shape[2]
    for h in range(_HB):
        o_ref[pl.ds(h * W, W)] = jax.lax.dot_general(
            x_ref[:, h, :].astype(jnp.bfloat16),
            eye_ref[...],
            (((0,), (0,)), ((), ())),
            preferred_element_type=jnp.float32,
        )


def _transpose(x):
    B, C, H, W = x.shape
    L = H * W
    eye = jnp.eye(C, dtype=jnp.bfloat16)
    return pl.pallas_call(
        _transpose_body,
        grid=(B, H // _HB),
        in_specs=[
            pl.BlockSpec((None, C, _HB, W), lambda b, k: (b, 0, k, 0)),
            pl.BlockSpec((C, C), lambda b, k: (0, 0)),
        ],
        out_specs=pl.BlockSpec((None, _HB * W, C), lambda b, k: (b, k, 0)),
        out_shape=jax.ShapeDtypeStruct((B, L, C), x.dtype),
    )(x, eye)


def _make_sc_gather(B, L, C, nchunk, ch):
    mesh = plsc.VectorSubcoreMesh(
        core_axis_name="c",
        subcore_axis_name="s",
        num_cores=_NUM_CORES,
        num_subcores=_NUM_SUBCORES,
    )
    rpw = nchunk * ch

    @functools.partial(
        pl.kernel,
        out_type=jax.ShapeDtypeStruct((B, L, C), jnp.float32),
        mesh=mesh,
        compiler_params=pltpu.CompilerParams(use_tc_tiling_on_sc=False),
        scratch_types=[
            pltpu.VMEM((nchunk * ch,), jnp.int32),
            pltpu.VMEM((ch, C), jnp.float32),
            pltpu.VMEM((ch, C), jnp.float32),
            pltpu.VMEM((ch, C), jnp.float32),
            pltpu.VMEM((ch, C), jnp.float32),
            pltpu.SemaphoreType.DMA,
            pltpu.SemaphoreType.DMA,
            pltpu.SemaphoreType.DMA,
            pltpu.SemaphoreType.DMA,
            pltpu.SemaphoreType.DMA,
            pltpu.SemaphoreType.DMA,
            pltpu.SemaphoreType.DMA,
            pltpu.SemaphoreType.DMA,
        ],
    )
    def sc_gather(xt_hbm, idx_hbm, out_hbm, idx_v,
                  buf0, buf1, buf2, buf3,
                  gs0, gs1, gs2, gs3, os0, os1, os2, os3):
        wid = lax.axis_index("s") * _NUM_CORES + lax.axis_index("c")
        pltpu.sync_copy(idx_hbm.at[wid], idx_v)

        bufs = (buf0, buf1, buf2, buf3)
        gsems = (gs0, gs1, gs2, gs3)
        osems = (os0, os1, os2, os3)
        base = wid * rpw
        ntot = B * nchunk
        gcp = [None] * 4
        ocp = [None] * 4

        def start_out(j):
            b, k = divmod(j, nchunk)
            bj = j % 4
            gcp[bj].wait()
            ocp[bj] = pltpu.async_copy(
                bufs[bj], out_hbm.at[b].at[pl.ds(base + k * ch, ch)], osems[bj]
            )

        for i in range(ntot):
            b, k = divmod(i, nchunk)
            bi = i % 4
            if ocp[bi] is not None:
                ocp[bi].wait()
            gcp[bi] = pltpu.async_copy(
                xt_hbm.at[b].at[idx_v.at[pl.ds(k * ch, ch)]], bufs[bi], gsems[bi]
            )
            if i >= 2:
                start_out(i - 2)
        start_out(ntot - 2)
        start_out(ntot - 1)
        for j in range(ntot - 4, ntot):
            ocp[j % 4].wait()

    return sc_gather


def kernel(x, hilbert_indices):
    B, C, H, W = x.shape
    L = H * W
    idx = hilbert_indices.astype(jnp.int32)

    xt = _transpose(x)

    nchunk, ch = 14, 112
    assert _NW * nchunk * ch == L
    idx_r = idx.reshape(_NW, nchunk * ch)
    return _make_sc_gather(B, L, C, nchunk, ch)(xt, idx_r)


# speedup vs baseline: 1.3766x; 1.3766x over previous
import functools

import jax
import jax.numpy as jnp
from jax import lax
from jax.experimental import pallas as pl
from jax.experimental.pallas import tpu as pltpu
from jax.experimental.pallas import tpu_sc as plsc

_NUM_CORES = 2
_NUM_SUBCORES = 16
_NW = _NUM_CORES * _NUM_SUBCORES

_HB = 8


def _transpose_body(x_ref, eye_a_ref, eye_b_ref, oa_ref, ob_ref):
    W = x_ref.shape[2]
    for h in range(_HB):
        xs = x_ref[:, h, :]
        oa_ref[pl.ds(h * W, W)] = jax.lax.dot_general(
            xs, eye_a_ref[...], (((0,), (0,)), ((), ())),
            preferred_element_type=jnp.float32,
        )
        ob_ref[pl.ds(h * W, W)] = jax.lax.dot_general(
            xs, eye_b_ref[...], (((0,), (0,)), ((), ())),
            preferred_element_type=jnp.float32,
        )


def _transpose(x):
    B, C, H, W = x.shape
    L = H * W
    eye_a = jnp.eye(C, 128, dtype=jnp.float32)
    eye_b = jnp.eye(C, 128, k=-128, dtype=jnp.float32)
    shp = jax.ShapeDtypeStruct((B, L, 128), x.dtype)
    return pl.pallas_call(
        _transpose_body,
        grid=(B, H // _HB),
        in_specs=[
            pl.BlockSpec((None, C, _HB, W), lambda b, k: (b, 0, k, 0)),
            pl.BlockSpec((C, 128), lambda b, k: (0, 0)),
            pl.BlockSpec((C, 128), lambda b, k: (0, 0)),
        ],
        out_specs=[
            pl.BlockSpec((None, _HB * W, 128), lambda b, k: (b, k, 0)),
            pl.BlockSpec((None, _HB * W, 128), lambda b, k: (b, k, 0)),
        ],
        out_shape=[shp, shp],
    )(x, eye_a, eye_b)


def _make_sc_gather(B, L, C, nchunk, ch):
    mesh = plsc.VectorSubcoreMesh(
        core_axis_name="c",
        subcore_axis_name="s",
        num_cores=_NUM_CORES,
        num_subcores=_NUM_SUBCORES,
    )
    rpw = nchunk * ch
    cb = C - 128

    @functools.partial(
        pl.kernel,
        out_type=jax.ShapeDtypeStruct((B, L, C), jnp.float32),
        mesh=mesh,
        compiler_params=pltpu.CompilerParams(use_tc_tiling_on_sc=False),
        scratch_types=[
            pltpu.VMEM((nchunk * ch,), jnp.int32),
            pltpu.VMEM((ch, 128), jnp.float32),
            pltpu.VMEM((ch, 128), jnp.float32),
            pltpu.VMEM((ch, 128), jnp.float32),
            pltpu.VMEM((ch, 128), jnp.float32),
            pltpu.SemaphoreType.DMA,
            pltpu.SemaphoreType.DMA,
            pltpu.SemaphoreType.DMA,
            pltpu.SemaphoreType.DMA,
        ],
    )
    def sc_gather(xta_hbm, xtb_hbm, idx_hbm, out_hbm, idx_v,
                  bufa0, bufa1, bufb0, bufb1, sa0, sa1, sb0, sb1):
        wid = lax.axis_index("s") * _NUM_CORES + lax.axis_index("c")
        pltpu.sync_copy(idx_hbm.at[wid], idx_v)

        bufas = (bufa0, bufa1)
        bufbs = (bufb0, bufb1)
        sas = (sa0, sa1)
        sbs = (sb0, sb1)
        base = wid * rpw
        prev = None
        for i in range(B * nchunk):
            b, k = divmod(i, nchunk)
            bi = i % 2
            ids = idx_v.at[pl.ds(k * ch, ch)]
            cpa = pltpu.async_copy(xta_hbm.at[b].at[ids], bufas[bi], sas[bi])
            cpb = pltpu.async_copy(xtb_hbm.at[b].at[ids], bufbs[bi], sbs[bi])
            if prev is not None:
                pa, pb_, pbi, pb_b, pb_row = prev
                dst = out_hbm.at[pb_b].at[pl.ds(pb_row, ch)]
                pa.wait()
                pltpu.sync_copy(bufas[pbi], dst.at[:, pl.ds(0, 128)])
                pb_.wait()
                pltpu.sync_copy(
                    bufbs[pbi].at[:, pl.ds(0, cb)], dst.at[:, pl.ds(128, cb)]
                )
            prev = (cpa, cpb, bi, b, base + k * ch)
        pa, pb_, pbi, pb_b, pb_row = prev
        dst = out_hbm.at[pb_b].at[pl.ds(pb_row, ch)]
        pa.wait()
        pltpu.sync_copy(bufas[pbi], dst.at[:, pl.ds(0, 128)])
        pb_.wait()
        pltpu.sync_copy(bufbs[pbi].at[:, pl.ds(0, cb)], dst.at[:, pl.ds(128, cb)])

    return sc_gather


def kernel(x, hilbert_indices):
    B, C, H, W = x.shape
    L = H * W
    idx = hilbert_indices.astype(jnp.int32)

    xta, xtb = _transpose(x)

    nchunk, ch = 7, 224
    assert _NW * nchunk * ch == L
    idx_r = idx.reshape(_NW, nchunk * ch)
    return _make_sc_gather(B, L, C, nchunk, ch)(xta, xtb, idx_r)


# speedup vs baseline: 1.4676x; 1.0661x over previous
import functools

import jax
import jax.numpy as jnp
from jax import lax
from jax.experimental import pallas as pl
from jax.experimental.pallas import tpu as pltpu
from jax.experimental.pallas import tpu_sc as plsc

_NUM_CORES = 2
_NUM_SUBCORES = 16
_NW = _NUM_CORES * _NUM_SUBCORES

_HB = 16


def _transpose_body(x_ref, eye_a_ref, eye_b_ref, oa_ref, ob_ref):
    W = x_ref.shape[2]
    for h in range(_HB):
        xs = x_ref[:, h, :]
        oa_ref[pl.ds(h * W, W)] = jax.lax.dot_general(
            xs, eye_a_ref[...], (((0,), (0,)), ((), ())),
            preferred_element_type=jnp.float32,
        )
        ob_ref[pl.ds(h * W, W)] = jax.lax.dot_general(
            xs, eye_b_ref[...], (((0,), (0,)), ((), ())),
            preferred_element_type=jnp.float32,
        )


def _transpose(x):
    B, C, H, W = x.shape
    L = H * W
    eye_a = jnp.eye(C, 128, dtype=jnp.float32)
    eye_b = jnp.eye(C, 128, k=-128, dtype=jnp.float32)
    shp = jax.ShapeDtypeStruct((B, L, 128), x.dtype)
    return pl.pallas_call(
        _transpose_body,
        grid=(B, H // _HB),
        in_specs=[
            pl.BlockSpec((None, C, _HB, W), lambda b, k: (b, 0, k, 0)),
            pl.BlockSpec((C, 128), lambda b, k: (0, 0)),
            pl.BlockSpec((C, 128), lambda b, k: (0, 0)),
        ],
        out_specs=[
            pl.BlockSpec((None, _HB * W, 128), lambda b, k: (b, k, 0)),
            pl.BlockSpec((None, _HB * W, 128), lambda b, k: (b, k, 0)),
        ],
        out_shape=[shp, shp],
    )(x, eye_a, eye_b)


def _make_sc_gather(B, L, C, nchunk, ch):
    mesh = plsc.VectorSubcoreMesh(
        core_axis_name="c",
        subcore_axis_name="s",
        num_cores=_NUM_CORES,
        num_subcores=_NUM_SUBCORES,
    )
    rpw = nchunk * ch
    cb = C - 128

    @functools.partial(
        pl.kernel,
        out_type=jax.ShapeDtypeStruct((B, L, C), jnp.float32),
        mesh=mesh,
        compiler_params=pltpu.CompilerParams(use_tc_tiling_on_sc=False),
        scratch_types=[
            pltpu.VMEM((nchunk * ch,), jnp.int32),
            pltpu.VMEM((nchunk * ch,), jnp.int32),
            pltpu.VMEM((ch, 128), jnp.float32),
            pltpu.VMEM((ch, 128), jnp.float32),
            pltpu.VMEM((ch, cb), jnp.float32),
            pltpu.VMEM((ch, cb), jnp.float32),
            pltpu.SemaphoreType.DMA,
            pltpu.SemaphoreType.DMA,
            pltpu.SemaphoreType.DMA,
            pltpu.SemaphoreType.DMA,
        ],
    )
    def sc_gather(xta_hbm, xtb_hbm, idx_hbm, idx2_hbm, out_hbm, idx_v, idx2_v,
                  bufa0, bufa1, bufb0, bufb1, sa0, sa1, sb0, sb1):
        wid = lax.axis_index("s") * _NUM_CORES + lax.axis_index("c")
        pltpu.sync_copy(idx_hbm.at[wid], idx_v)
        pltpu.sync_copy(idx2_hbm.at[wid], idx2_v)

        bufas = (bufa0, bufa1)
        bufbs = (bufb0, bufb1)
        sas = (sa0, sa1)
        sbs = (sb0, sb1)
        base = wid * rpw
        prev = None
        for i in range(B * nchunk):
            b, k = divmod(i, nchunk)
            bi = i % 2
            ids = idx_v.at[pl.ds(k * ch, ch)]
            ids2 = idx2_v.at[pl.ds(k * ch, ch)]
            cpa = pltpu.async_copy(xta_hbm.at[b].at[ids], bufas[bi], sas[bi])
            cpb = pltpu.async_copy(xtb_hbm.at[b].at[ids2], bufbs[bi], sbs[bi])
            if prev is not None:
                pa, pb_, pbi, pb_b, pb_row = prev
                dst = out_hbm.at[pb_b].at[pl.ds(pb_row, ch)]
                pa.wait()
                pltpu.sync_copy(bufas[pbi], dst.at[:, pl.ds(0, 128)])
                pb_.wait()
                pltpu.sync_copy(bufbs[pbi], dst.at[:, pl.ds(128, cb)])
            prev = (cpa, cpb, bi, b, base + k * ch)
        pa, pb_, pbi, pb_b, pb_row = prev
        dst = out_hbm.at[pb_b].at[pl.ds(pb_row, ch)]
        pa.wait()
        pltpu.sync_copy(bufas[pbi], dst.at[:, pl.ds(0, 128)])
        pb_.wait()
        pltpu.sync_copy(bufbs[pbi], dst.at[:, pl.ds(128, cb)])

    return sc_gather


def kernel(x, hilbert_indices):
    B, C, H, W = x.shape
    L = H * W
    idx = hilbert_indices.astype(jnp.int32)

    xta, xtb = _transpose(x)

    nchunk, ch = 7, 224
    assert _NW * nchunk * ch == L
    idx_r = idx.reshape(_NW, nchunk * ch)
    xtb2 = xtb.reshape(B, 2 * L, 64)
    idx2_r = (idx * 2).reshape(_NW, nchunk * ch)
    return _make_sc_gather(B, L, C, nchunk, ch)(xta, xtb2, idx_r, idx2_r)
